# baseline (speedup 1.0000x reference)
import jax
import jax.numpy as jnp
from jax.experimental import pallas as pl

_TILE = 2048


def _router_kernel(x_ref, wcat_ref, bcat_ref, w2_ref, b2_ref, w3_ref, b3_ref,
                   wv2_ref, bv2_ref, logits_ref, values_ref):
    x = x_ref[...]
    h_all = jnp.dot(x, wcat_ref[...], preferred_element_type=jnp.float32)
    h_all = jnp.maximum(h_all + bcat_ref[...], 0.0)
    h = h_all[:, :384]
    v = h_all[:, 384:]
    h2 = jnp.dot(h, w2_ref[...], preferred_element_type=jnp.float32)
    h2 = jnp.maximum(h2 + b2_ref[...], 0.0)
    logits_ref[...] = (
        jnp.dot(h2, w3_ref[...], preferred_element_type=jnp.float32)
        + b3_ref[...]
    )
    values_ref[...] = (
        jnp.dot(v, wv2_ref[...], preferred_element_type=jnp.float32)
        + bv2_ref[...]
    )


def kernel(hidden_states, W1, b1, W2, b2, W3, b3, Wv1, bv1, Wv2, bv2):
    B, S, H = hidden_states.shape
    N = B * S
    E = W3.shape[1]
    flat = hidden_states.reshape(N, H)
    Wcat = jnp.concatenate([W1, Wv1], axis=1)
    bcat = jnp.concatenate([b1, bv1])[None, :]
    logits, values = pl.pallas_call(
        _router_kernel,
        grid=(N // _TILE,),
        in_specs=[
            pl.BlockSpec((_TILE, H), lambda i: (i, 0)),
            pl.BlockSpec((H, H), lambda i: (0, 0)),
            pl.BlockSpec((1, H), lambda i: (0, 0)),
            pl.BlockSpec((H // 2, H // 4), lambda i: (0, 0)),
            pl.BlockSpec((1, H // 4), lambda i: (0, 0)),
            pl.BlockSpec((H // 4, E), lambda i: (0, 0)),
            pl.BlockSpec((1, E), lambda i: (0, 0)),
            pl.BlockSpec((H // 2, 1), lambda i: (0, 0)),
            pl.BlockSpec((1, 1), lambda i: (0, 0)),
        ],
        out_specs=[
            pl.BlockSpec((_TILE, E), lambda i: (i, 0)),
            pl.BlockSpec((_TILE, 1), lambda i: (i, 0)),
        ],
        out_shape=[
            jax.ShapeDtypeStruct((N, E), jnp.float32),
            jax.ShapeDtypeStruct((N, 1), jnp.float32),
        ],
    )(flat, Wcat, bcat, W2, b2[None, :], W3, b3[None, :], Wv2, bv2[None, :])
    return (logits.reshape(B, S, E), values.reshape(B, S, 1))


# speedup vs baseline: 1.0014x; 1.0014x over previous
import jax
import jax.numpy as jnp
from jax.experimental import pallas as pl
from jax.experimental.pallas import tpu as pltpu

_TILE = 2048


def _router_kernel(x_ref, wcat_ref, bcat_ref, w2_ref, b2_ref, w3_ref, b3_ref,
                   wv2_ref, bv2_ref, logits_ref, values_ref):
    x = x_ref[...]
    h_all = jnp.dot(x, wcat_ref[...], preferred_element_type=jnp.float32)
    h_all = jnp.maximum(h_all + bcat_ref[...], 0.0)
    h = h_all[:, :384]
    v = h_all[:, 384:]
    h2 = jnp.dot(h, w2_ref[...], preferred_element_type=jnp.float32)
    h2 = jnp.maximum(h2 + b2_ref[...], 0.0)
    logits_ref[...] = (
        jnp.dot(h2, w3_ref[...], preferred_element_type=jnp.float32)
        + b3_ref[...]
    )
    values_ref[...] = (
        jnp.dot(v, wv2_ref[...], preferred_element_type=jnp.float32)
        + bv2_ref[...]
    )


def kernel(hidden_states, W1, b1, W2, b2, W3, b3, Wv1, bv1, Wv2, bv2):
    B, S, H = hidden_states.shape
    N = B * S
    E = W3.shape[1]
    flat = hidden_states.reshape(N, H)
    Wcat = jnp.concatenate([W1, Wv1], axis=1)
    bcat = jnp.concatenate([b1, bv1])[None, :]
    logits, values = pl.pallas_call(
        _router_kernel,
        grid=(N // _TILE,),
        compiler_params=pltpu.CompilerParams(
            dimension_semantics=("parallel",),
        ),
        in_specs=[
            pl.BlockSpec((_TILE, H), lambda i: (i, 0)),
            pl.BlockSpec((H, H), lambda i: (0, 0)),
            pl.BlockSpec((1, H), lambda i: (0, 0)),
            pl.BlockSpec((H // 2, H // 4), lambda i: (0, 0)),
            pl.BlockSpec((1, H // 4), lambda i: (0, 0)),
            pl.BlockSpec((H // 4, E), lambda i: (0, 0)),
            pl.BlockSpec((1, E), lambda i: (0, 0)),
            pl.BlockSpec((H // 2, 1), lambda i: (0, 0)),
            pl.BlockSpec((1, 1), lambda i: (0, 0)),
        ],
        out_specs=[
            pl.BlockSpec((_TILE, E), lambda i: (i, 0)),
            pl.BlockSpec((_TILE, 1), lambda i: (i, 0)),
        ],
        out_shape=[
            jax.ShapeDtypeStruct((N, E), jnp.float32),
            jax.ShapeDtypeStruct((N, 1), jnp.float32),
        ],
    )(flat, Wcat, bcat, W2, b2[None, :], W3, b3[None, :], Wv2, bv2[None, :])
    return (logits.reshape(B, S, E), values.reshape(B, S, 1))
